# baseline (speedup 1.0000x reference)
import functools

import jax
import jax.numpy as jnp
from jax import lax
from jax.experimental import pallas as pl
from jax.experimental.pallas import tpu as pltpu
from jax.experimental.pallas import tpu_sc as plsc

f32 = jnp.float32
i32 = jnp.int32

NC = 2
NS = 16
NW = NC * NS
L = 16
CHUNK = 128
ACHUNK = 64

_mesh = plsc.VectorSubcoreMesh(core_axis_name="c", subcore_axis_name="s")
_params = pltpu.CompilerParams(needs_layout_passes=False)


def _fill_1d(ref, n, value, dtype):
    v = jnp.full((L,), value, dtype)

    @pl.loop(0, n // L)
    def _(j):
        ref[pl.ds(j * L, L)] = v


def _zero_2d(ref, rows, cols):
    z = jnp.zeros((L,), f32)

    @pl.loop(0, rows)
    def _(r):
        for cv in range(cols // L):
            ref[r, pl.ds(cv * L, L)] = z


def _scatter_max(tab, idx16, val16):
    cur = plsc.load_gather(tab, [idx16])
    new = jnp.maximum(cur, val16)
    plsc.store_scatter(tab, [idx16], new)
    chk = plsc.load_gather(tab, [idx16])
    need = chk < new

    def cond(need):
        return jnp.any(need)

    def body(need):
        plsc.store_scatter(tab, [idx16], new, mask=need)
        chk = plsc.load_gather(tab, [idx16])
        return chk < new

    lax.while_loop(cond, body, need)


def _sc_merge_tables(part, spm, mergebuf, accv, out, npad, op):
    scid = lax.axis_index("c")
    sid = lax.axis_index("s")
    sl = npad // NS
    pltpu.sync_copy(part, spm.at[sid])
    plsc.subcore_barrier()
    for k in range(NS):
        pltpu.sync_copy(spm.at[k, pl.ds(sid * sl, sl)], mergebuf.at[k])

    @pl.loop(0, sl // L)
    def _(j):
        s = pl.ds(j * L, L)
        m = mergebuf[0, s]
        for k in range(1, NS):
            m = op(m, mergebuf[k, s])
        accv[s] = m

    pltpu.sync_copy(accv, out.at[scid, pl.ds(sid * sl, sl)])


def _worker_id():
    return lax.axis_index("s") * NC + lax.axis_index("c")


def _k1_body(nedge, npad, nrel,
             emb, head, tail, etype, rel,
             s_out, mtab_out,
             hidx0, hidx1, tidx0, tidx1, et0, et1, sbuf,
             hrows0, hrows1, trows0, trows1, relv,
             mtab, mergebuf, accv, spm,
             semI0, semI1, semR0, semR1):
    ck = CHUNK
    nch = nedge // ck
    tpw = pl.cdiv(nch, NW)
    T = tpw + (tpw % 2)
    wid = _worker_id()
    hidx = (hidx0, hidx1)
    tidx = (tidx0, tidx1)
    et = (et0, et1)
    hrows = (hrows0, hrows1)
    trows = (trows0, trows1)
    semI = (semI0, semI1)
    semR = (semR0, semR1)

    pltpu.sync_copy(rel, relv)
    _fill_1d(mtab, npad, -jnp.inf, f32)
    lanes = jnp.arange(L, dtype=i32)

    def chunk_of(g):
        c = wid + g * NW
        return jnp.minimum(c, nch - 1), c < nch

    def issue_idx(g, b):
        c, _ = chunk_of(g)
        off = c * ck
        pltpu.async_copy(head.at[pl.ds(off, ck)], hidx[b], semI[b])
        pltpu.async_copy(tail.at[pl.ds(off, ck)], tidx[b], semI[b])
        pltpu.async_copy(etype.at[pl.ds(off, ck)], et[b], semI[b])

    def drain_idx(b):
        pltpu.make_async_copy(head.at[pl.ds(0, ck)], hidx[b], semI[b]).wait()
        pltpu.make_async_copy(tail.at[pl.ds(0, ck)], tidx[b], semI[b]).wait()
        pltpu.make_async_copy(etype.at[pl.ds(0, ck)], et[b], semI[b]).wait()

    def issue_rows(b):
        pltpu.async_copy(emb.at[hidx[b]], hrows[b], semR[b])
        pltpu.async_copy(emb.at[tidx[b]], trows[b], semR[b])

    def drain_rows(b):
        pltpu.make_async_copy(emb.at[hidx[b]], hrows[b], semR[b]).wait()
        pltpu.make_async_copy(emb.at[tidx[b]], trows[b], semR[b]).wait()

    def compute(g, b):
        c, valid = chunk_of(g)

        @pl.when(valid)
        def _():
            @pl.loop(0, ck // L)
            def _(i):
                io = i * L
                heads = hidx[b][pl.ds(io, L)]
                et16 = et[b][pl.ds(io, L)]
                rrow = jnp.where(et16 == 0, nrel - 1, et16 - 1)
                s16 = jnp.zeros((L,), f32)
                for j in range(L):
                    e = io + j
                    rr = rrow[j]
                    parts = []
                    for p in range(4):
                        cs0 = pl.ds((2 * p) * L, L)
                        cs1 = pl.ds((2 * p + 1) * L, L)
                        a0 = hrows[b][e, cs0] * relv[rr, cs0] * trows[b][e, cs0]
                        a1 = hrows[b][e, cs1] * relv[rr, cs1] * trows[b][e, cs1]
                        parts.append(a0 + a1)
                    d = jnp.sum((parts[0] + parts[1]) + (parts[2] + parts[3]))
                    s16 = jnp.where(lanes == j, d, s16)
                sv = jnp.exp(s16)
                sbuf[pl.ds(io, L)] = sv
                _scatter_max(mtab, heads, sv)

            pltpu.sync_copy(sbuf, s_out.at[pl.ds(c * ck, ck)])

    issue_idx(0, 0)
    drain_idx(0)
    issue_rows(0)
    issue_idx(1, 1)

    @pl.loop(0, T, step=2)
    def _(g2):
        for b in range(2):
            g = g2 + b
            drain_rows(b)
            drain_idx(1 - b)
            issue_rows(1 - b)
            compute(g, b)
            issue_idx(g + 2, b)

    drain_rows(0)
    drain_idx(1)

    _sc_merge_tables(mtab, spm, mergebuf, accv, mtab_out, npad, jnp.maximum)


def _k1(emb, head, tail, etype, rel, npad):
    nedge = head.shape[0]
    nrel = rel.shape[0]
    sl = npad // NS
    kfn = pl.kernel(
        functools.partial(_k1_body, nedge, npad, nrel),
        out_type=(jax.ShapeDtypeStruct((nedge,), f32),
                  jax.ShapeDtypeStruct((NC, npad), f32)),
        mesh=_mesh,
        compiler_params=_params,
        scratch_types=[
            pltpu.VMEM((CHUNK,), i32), pltpu.VMEM((CHUNK,), i32),
            pltpu.VMEM((CHUNK,), i32), pltpu.VMEM((CHUNK,), i32),
            pltpu.VMEM((CHUNK,), i32), pltpu.VMEM((CHUNK,), i32),
            pltpu.VMEM((CHUNK,), f32),
            pltpu.VMEM((CHUNK, 128), f32), pltpu.VMEM((CHUNK, 128), f32),
            pltpu.VMEM((CHUNK, 128), f32), pltpu.VMEM((CHUNK, 128), f32),
            pltpu.VMEM((nrel, 128), f32),
            pltpu.VMEM((npad,), f32),
            pltpu.VMEM((NS, sl), f32),
            pltpu.VMEM((sl,), f32),
            pltpu.VMEM_SHARED((NS, npad), f32),
            pltpu.SemaphoreType.DMA, pltpu.SemaphoreType.DMA,
            pltpu.SemaphoreType.DMA, pltpu.SemaphoreType.DMA,
        ],
    )
    return kfn(emb, head, tail, etype, rel)


def _k2_body(nedge, npad,
             s_in, head, mtab_in,
             ex_out, ztab_out,
             hidx, sbuf, exbuf, mvec, tmpv, ztab, mergebuf, accv, spm, sem):
    ck = CHUNK
    nch = nedge // ck
    tpw = pl.cdiv(nch, NW)
    wid = _worker_id()
    pltpu.sync_copy(mtab_in.at[0], mvec)
    pltpu.sync_copy(mtab_in.at[1], tmpv)

    @pl.loop(0, npad // L)
    def _(j):
        s = pl.ds(j * L, L)
        mvec[s] = jnp.maximum(mvec[s], tmpv[s])

    _fill_1d(ztab, npad, 0.0, f32)

    @pl.loop(0, tpw)
    def _(t):
        c = wid + t * NW

        @pl.when(c < nch)
        def _():
            off = c * ck
            pltpu.sync_copy(head.at[pl.ds(off, ck)], hidx)
            pltpu.sync_copy(s_in.at[pl.ds(off, ck)], sbuf)
            for i in range(ck // L):
                heads = hidx[pl.ds(i * L, L)]
                s16 = sbuf[pl.ds(i * L, L)]
                mh = plsc.load_gather(mvec, [heads])
                ex16 = jnp.exp(s16 - mh)
                exbuf[pl.ds(i * L, L)] = ex16
                plsc.addupdate_scatter(ztab, [heads], ex16)
            pltpu.sync_copy(exbuf, ex_out.at[pl.ds(off, ck)])

    _sc_merge_tables(ztab, spm, mergebuf, accv, ztab_out, npad, jnp.add)


def _k2(s, head, mtab, npad):
    nedge = head.shape[0]
    sl = npad // NS
    kfn = pl.kernel(
        functools.partial(_k2_body, nedge, npad),
        out_type=(jax.ShapeDtypeStruct((nedge,), f32),
                  jax.ShapeDtypeStruct((NC, npad), f32)),
        mesh=_mesh,
        compiler_params=_params,
        scratch_types=[
            pltpu.VMEM((CHUNK,), i32),
            pltpu.VMEM((CHUNK,), f32),
            pltpu.VMEM((CHUNK,), f32),
            pltpu.VMEM((npad,), f32),
            pltpu.VMEM((npad,), f32),
            pltpu.VMEM((npad,), f32),
            pltpu.VMEM((NS, sl), f32),
            pltpu.VMEM((sl,), f32),
            pltpu.VMEM_SHARED((NS, npad), f32),
            pltpu.SemaphoreType.DMA,
        ],
    )
    return kfn(s, head, mtab)


def _agg_body(nedge, npad, with_attn, *refs):
    ck = ACHUNK
    if with_attn:
        (emb, head, tail, ex_in, ztab_in, agg_out,
         hidx0, hidx1, tidx0, tidx1, wbuf0, wbuf1, sidx,
         zvec, trows0, trows1, orows, spm,
         semI0, semI1, semR0, semR1, semS) = refs
    else:
        (emb, head, tail, ex_in, agg_out,
         hidx0, hidx1, tidx0, tidx1, wbuf0, wbuf1, sidx,
         trows0, trows1, orows, spm,
         semI0, semI1, semR0, semR1, semS) = refs
    hidx = (hidx0, hidx1)
    tidx = (tidx0, tidx1)
    wbuf = (wbuf0, wbuf1)
    trows = (trows0, trows1)
    semI = (semI0, semI1)
    semR = (semR0, semR1)

    nch = nedge // ck
    tpw = pl.cdiv(nch, NW)
    T = tpw + (tpw % 2)
    wid = _worker_id()
    scid = lax.axis_index("c")
    sid = lax.axis_index("s")
    sl = npad // NS
    lanes = jnp.arange(L, dtype=i32)

    if with_attn:
        pltpu.sync_copy(ztab_in.at[0], zvec)

        @pl.loop(0, npad // ck)
        def _(k):
            pltpu.sync_copy(ztab_in.at[1, pl.ds(k * ck, ck)], wbuf0)
            for cv in range(ck // L):
                d = pl.ds(k * ck + cv * L, L)
                zvec[d] = zvec[d] + wbuf0[pl.ds(cv * L, L)]

    _zero_2d(orows, ck, 128)
    for k in range(sl // ck):
        pltpu.sync_copy(orows, spm.at[pl.ds(sid * sl + k * ck, ck)])
    plsc.subcore_barrier()

    def chunk_of(g):
        c = wid + g * NW
        return jnp.minimum(c, nch - 1), c < nch

    def issue_idx(g, b):
        c, _ = chunk_of(g)
        off = c * ck
        pltpu.async_copy(head.at[pl.ds(off, ck)], hidx[b], semI[b])
        pltpu.async_copy(tail.at[pl.ds(off, ck)], tidx[b], semI[b])
        pltpu.async_copy(ex_in.at[pl.ds(off, ck)], wbuf[b], semI[b])

    def drain_idx(b):
        pltpu.make_async_copy(head.at[pl.ds(0, ck)], hidx[b], semI[b]).wait()
        pltpu.make_async_copy(tail.at[pl.ds(0, ck)], tidx[b], semI[b]).wait()
        pltpu.make_async_copy(ex_in.at[pl.ds(0, ck)], wbuf[b], semI[b]).wait()

    def issue_rows(b):
        pltpu.async_copy(emb.at[tidx[b]], trows[b], semR[b])

    def drain_rows(b):
        pltpu.make_async_copy(emb.at[tidx[b]], trows[b], semR[b]).wait()

    def drain_scatter():
        pltpu.make_async_copy(orows, spm.at[sidx], semS).wait()

    def compute(g, b):
        c, valid = chunk_of(g)
        prev_valid = jnp.logical_and(g >= 1, (wid + (g - 1) * NW) < nch)

        @pl.when(prev_valid)
        def _():
            drain_scatter()

        @pl.when(valid)
        def _():
            for j in range(ck // L):
                s = pl.ds(j * L, L)
                sidx[s] = hidx[b][s]

            @pl.loop(0, ck // L)
            def _(i):
                io = i * L
                w16 = wbuf[b][pl.ds(io, L)]
                if with_attn:
                    heads = hidx[b][pl.ds(io, L)]
                    zh = plsc.load_gather(zvec, [heads])
                    w16 = w16 / zh
                for j in range(L):
                    e = io + j
                    a = jnp.full((L,), w16[j], f32)
                    for cv in range(128 // L):
                        cs = pl.ds(cv * L, L)
                        orows[e, cs] = trows[b][e, cs] * a

            pltpu.async_copy(orows, spm.at[sidx], semS, add=True)

    issue_idx(0, 0)
    drain_idx(0)
    issue_rows(0)
    issue_idx(1, 1)

    @pl.loop(0, T, step=2)
    def _(g2):
        for b in range(2):
            g = g2 + b
            drain_rows(b)
            drain_idx(1 - b)
            issue_rows(1 - b)
            compute(g, b)
            issue_idx(g + 2, b)

    drain_rows(0)
    drain_idx(1)
    last_valid = (wid + (T - 1) * NW) < nch

    @pl.when(last_valid)
    def _():
        drain_scatter()

    plsc.subcore_barrier()
    pltpu.sync_copy(spm.at[pl.ds(sid * sl, sl)],
                    agg_out.at[scid, pl.ds(sid * sl, sl)])


def _k3(emb, head, tail, ex, ztab, npad):
    nedge = head.shape[0]
    kfn = pl.kernel(
        functools.partial(_agg_body, nedge, npad, True),
        out_type=jax.ShapeDtypeStruct((NC, npad, 128), f32),
        mesh=_mesh,
        compiler_params=_params,
        scratch_types=[
            pltpu.VMEM((ACHUNK,), i32), pltpu.VMEM((ACHUNK,), i32),
            pltpu.VMEM((ACHUNK,), i32), pltpu.VMEM((ACHUNK,), i32),
            pltpu.VMEM((ACHUNK,), f32), pltpu.VMEM((ACHUNK,), f32),
            pltpu.VMEM((ACHUNK,), i32),
            pltpu.VMEM((npad,), f32),
            pltpu.VMEM((ACHUNK, 128), f32), pltpu.VMEM((ACHUNK, 128), f32),
            pltpu.VMEM((ACHUNK, 128), f32),
            pltpu.VMEM_SHARED((npad, 128), f32),
            pltpu.SemaphoreType.DMA, pltpu.SemaphoreType.DMA,
            pltpu.SemaphoreType.DMA, pltpu.SemaphoreType.DMA,
            pltpu.SemaphoreType.DMA,
        ],
    )
    return kfn(emb, head, tail, ex, ztab)


def _k5(emb, uidx, iidx, w, npad):
    nedge = uidx.shape[0]
    kfn = pl.kernel(
        functools.partial(_agg_body, nedge, npad, False),
        out_type=jax.ShapeDtypeStruct((NC, npad, 128), f32),
        mesh=_mesh,
        compiler_params=_params,
        scratch_types=[
            pltpu.VMEM((ACHUNK,), i32), pltpu.VMEM((ACHUNK,), i32),
            pltpu.VMEM((ACHUNK,), i32), pltpu.VMEM((ACHUNK,), i32),
            pltpu.VMEM((ACHUNK,), f32), pltpu.VMEM((ACHUNK,), f32),
            pltpu.VMEM((ACHUNK,), i32),
            pltpu.VMEM((ACHUNK, 128), f32), pltpu.VMEM((ACHUNK, 128), f32),
            pltpu.VMEM((ACHUNK, 128), f32),
            pltpu.VMEM_SHARED((npad, 128), f32),
            pltpu.SemaphoreType.DMA, pltpu.SemaphoreType.DMA,
            pltpu.SemaphoreType.DMA, pltpu.SemaphoreType.DMA,
            pltpu.SemaphoreType.DMA,
        ],
    )
    return kfn(emb, uidx, iidx, w)


def _norm_body(t_ref, o_ref):
    x = t_ref[0] + t_ref[1]
    n = jnp.sum(x * x, axis=1, keepdims=True)
    o_ref[...] = x / jnp.maximum(jnp.sqrt(n), 1e-12)


def _sum_norm(tabs, nrows):
    r = 200 if nrows % 200 == 0 else 8
    return pl.pallas_call(
        _norm_body,
        grid=(nrows // r,),
        in_specs=[pl.BlockSpec((NC, r, 128), lambda i: (0, i, 0))],
        out_specs=pl.BlockSpec((r, 128), lambda i: (i, 0)),
        out_shape=jax.ShapeDtypeStruct((nrows, 128), f32),
    )(tabs)


def kernel(user_emb, item_emb, edge_index, edge_type, inter_edge,
           inter_edge_w, relation_emb):
    n_ent = item_emb.shape[0]
    n_users = user_emb.shape[0]
    npad_e = pl.cdiv(n_ent + 1, NS * L) * NS * L

    head = edge_index[0]
    tail = edge_index[1]
    etype = edge_type
    ne = head.shape[0]
    ne_pad = pl.cdiv(ne, CHUNK) * CHUNK
    if ne_pad != ne:
        head = jnp.pad(head, (0, ne_pad - ne), constant_values=n_ent)
        tail = jnp.pad(tail, (0, ne_pad - ne), constant_values=0)
        etype = jnp.pad(etype, (0, ne_pad - ne), constant_values=0)

    emb = item_emb
    for _ in range(2):
        s, mtab = _k1(emb, head, tail, etype, relation_emb, npad_e)
        ex, ztab = _k2(s, head, mtab, npad_e)
        agg = _k3(emb, head, tail, ex, ztab, npad_e)
        emb = _sum_norm(agg, n_ent)

    uidx = inter_edge[0]
    iidx = inter_edge[1]
    w = inter_edge_w
    ni = uidx.shape[0]
    ni_pad = pl.cdiv(ni, CHUNK) * CHUNK
    npad_u = pl.cdiv(n_users + 1, NS * L) * NS * L
    if ni_pad != ni:
        uidx = jnp.pad(uidx, (0, ni_pad - ni), constant_values=n_users)
        iidx = jnp.pad(iidx, (0, ni_pad - ni), constant_values=0)
        w = jnp.pad(w, (0, ni_pad - ni), constant_values=0.0)

    uagg = _k5(emb, uidx, iidx, w, npad_u)
    user_out = _sum_norm(uagg, n_users)
    return (user_out, emb)


# speedup vs baseline: 1.0516x; 1.0516x over previous
import functools

import jax
import jax.numpy as jnp
from jax import lax
from jax.experimental import pallas as pl
from jax.experimental.pallas import tpu as pltpu
from jax.experimental.pallas import tpu_sc as plsc

f32 = jnp.float32
i32 = jnp.int32

NC = 2
NS = 16
NW = NC * NS
L = 16
CHUNK = 128
ACHUNK = 64

_mesh = plsc.VectorSubcoreMesh(core_axis_name="c", subcore_axis_name="s")
_params = pltpu.CompilerParams(needs_layout_passes=False)


def _fill_1d(ref, n, value, dtype):
    v = jnp.full((L,), value, dtype)

    @pl.loop(0, n // L)
    def _(j):
        ref[pl.ds(j * L, L)] = v


def _zero_2d(ref, rows, cols):
    z = jnp.zeros((L,), f32)

    @pl.loop(0, rows)
    def _(r):
        for cv in range(cols // L):
            ref[r, pl.ds(cv * L, L)] = z


def _scatter_max(tab, idx16, val16):
    cur = plsc.load_gather(tab, [idx16])
    new = jnp.maximum(cur, val16)
    plsc.store_scatter(tab, [idx16], new)
    chk = plsc.load_gather(tab, [idx16])
    need = chk < new

    def cond(need):
        return jnp.any(need)

    def body(need):
        plsc.store_scatter(tab, [idx16], new, mask=need)
        chk = plsc.load_gather(tab, [idx16])
        return chk < new

    lax.while_loop(cond, body, need)


def _sc_merge_tables(part, spm, mergebuf, accv, out, npad, op):
    scid = lax.axis_index("c")
    sid = lax.axis_index("s")
    sl = npad // NS
    pltpu.sync_copy(part, spm.at[sid])
    plsc.subcore_barrier()
    for k in range(NS):
        pltpu.sync_copy(spm.at[k, pl.ds(sid * sl, sl)], mergebuf.at[k])

    @pl.loop(0, sl // L)
    def _(j):
        s = pl.ds(j * L, L)
        m = mergebuf[0, s]
        for k in range(1, NS):
            m = op(m, mergebuf[k, s])
        accv[s] = m

    pltpu.sync_copy(accv, out.at[scid, pl.ds(sid * sl, sl)])


def _worker_id():
    return lax.axis_index("s") * NC + lax.axis_index("c")


def _k1_body(nedge, npad, nrel,
             emb, head, tail, etype, rel,
             s_out, mtab_out,
             hidx0, hidx1, tidx0, tidx1, et0, et1, sbuf,
             hrows0, hrows1, trows0, trows1, relv,
             mtab, mergebuf, accv, spm,
             semI0, semI1, semR0, semR1):
    ck = CHUNK
    nch = nedge // ck
    tpw = pl.cdiv(nch, NW)
    T = tpw + (tpw % 2)
    wid = _worker_id()
    hidx = (hidx0, hidx1)
    tidx = (tidx0, tidx1)
    et = (et0, et1)
    hrows = (hrows0, hrows1)
    trows = (trows0, trows1)
    semI = (semI0, semI1)
    semR = (semR0, semR1)

    pltpu.sync_copy(rel, relv)
    _fill_1d(mtab, npad, -jnp.inf, f32)
    lanes = jnp.arange(L, dtype=i32)

    def chunk_of(g):
        c = wid + g * NW
        return jnp.minimum(c, nch - 1), c < nch

    def issue_idx(g, b):
        c, _ = chunk_of(g)
        off = c * ck
        pltpu.async_copy(head.at[pl.ds(off, ck)], hidx[b], semI[b])
        pltpu.async_copy(tail.at[pl.ds(off, ck)], tidx[b], semI[b])
        pltpu.async_copy(etype.at[pl.ds(off, ck)], et[b], semI[b])

    def drain_idx(b):
        pltpu.make_async_copy(head.at[pl.ds(0, ck)], hidx[b], semI[b]).wait()
        pltpu.make_async_copy(tail.at[pl.ds(0, ck)], tidx[b], semI[b]).wait()
        pltpu.make_async_copy(etype.at[pl.ds(0, ck)], et[b], semI[b]).wait()

    def issue_rows(b):
        pltpu.async_copy(emb.at[hidx[b]], hrows[b], semR[b])
        pltpu.async_copy(emb.at[tidx[b]], trows[b], semR[b])

    def drain_rows(b):
        pltpu.make_async_copy(emb.at[hidx[b]], hrows[b], semR[b]).wait()
        pltpu.make_async_copy(emb.at[tidx[b]], trows[b], semR[b]).wait()

    def compute(g, b):
        c, valid = chunk_of(g)

        @pl.when(valid)
        def _():
            @pl.loop(0, ck // L)
            def _(i):
                io = i * L
                heads = hidx[b][pl.ds(io, L)]
                et16 = et[b][pl.ds(io, L)]
                rrow = jnp.where(et16 == 0, nrel - 1, et16 - 1)
                s16 = jnp.zeros((L,), f32)
                for j in range(L):
                    e = io + j
                    rr = rrow[j]
                    acc = jnp.zeros((L,), f32)
                    for cv in range(128 // L):
                        cs = pl.ds(cv * L, L)
                        acc = acc + (hrows[b][e, cs] * relv[rr, cs]
                                     * trows[b][e, cs])
                    d = jnp.sum(acc)
                    s16 = jnp.where(lanes == j, d, s16)
                sv = jnp.exp(s16)
                sbuf[pl.ds(io, L)] = sv
                _scatter_max(mtab, heads, sv)

            pltpu.sync_copy(sbuf, s_out.at[pl.ds(c * ck, ck)])

    issue_idx(0, 0)
    drain_idx(0)
    issue_rows(0)
    issue_idx(1, 1)

    @pl.loop(0, T, step=2)
    def _(g2):
        for b in range(2):
            g = g2 + b
            drain_rows(b)
            drain_idx(1 - b)
            issue_rows(1 - b)
            compute(g, b)
            issue_idx(g + 2, b)

    drain_rows(0)
    drain_idx(1)

    _sc_merge_tables(mtab, spm, mergebuf, accv, mtab_out, npad, jnp.maximum)


def _k1(emb, head, tail, etype, rel, npad):
    nedge = head.shape[0]
    nrel = rel.shape[0]
    sl = npad // NS
    kfn = pl.kernel(
        functools.partial(_k1_body, nedge, npad, nrel),
        out_type=(jax.ShapeDtypeStruct((nedge,), f32),
                  jax.ShapeDtypeStruct((NC, npad), f32)),
        mesh=_mesh,
        compiler_params=_params,
        scratch_types=[
            pltpu.VMEM((CHUNK,), i32), pltpu.VMEM((CHUNK,), i32),
            pltpu.VMEM((CHUNK,), i32), pltpu.VMEM((CHUNK,), i32),
            pltpu.VMEM((CHUNK,), i32), pltpu.VMEM((CHUNK,), i32),
            pltpu.VMEM((CHUNK,), f32),
            pltpu.VMEM((CHUNK, 128), f32), pltpu.VMEM((CHUNK, 128), f32),
            pltpu.VMEM((CHUNK, 128), f32), pltpu.VMEM((CHUNK, 128), f32),
            pltpu.VMEM((nrel, 128), f32),
            pltpu.VMEM((npad,), f32),
            pltpu.VMEM((NS, sl), f32),
            pltpu.VMEM((sl,), f32),
            pltpu.VMEM_SHARED((NS, npad), f32),
            pltpu.SemaphoreType.DMA, pltpu.SemaphoreType.DMA,
            pltpu.SemaphoreType.DMA, pltpu.SemaphoreType.DMA,
        ],
    )
    return kfn(emb, head, tail, etype, rel)


def _k2_body(nedge, npad,
             s_in, head, mtab_in,
             ex_out, ztab_out,
             hidx, sbuf, exbuf, mvec, tmpv, ztab, mergebuf, accv, spm, sem):
    ck = CHUNK
    nch = nedge // ck
    tpw = pl.cdiv(nch, NW)
    wid = _worker_id()
    pltpu.sync_copy(mtab_in.at[0], mvec)
    pltpu.sync_copy(mtab_in.at[1], tmpv)

    @pl.loop(0, npad // L)
    def _(j):
        s = pl.ds(j * L, L)
        mvec[s] = jnp.maximum(mvec[s], tmpv[s])

    _fill_1d(ztab, npad, 0.0, f32)

    @pl.loop(0, tpw)
    def _(t):
        c = wid + t * NW

        @pl.when(c < nch)
        def _():
            off = c * ck
            pltpu.sync_copy(head.at[pl.ds(off, ck)], hidx)
            pltpu.sync_copy(s_in.at[pl.ds(off, ck)], sbuf)
            for i in range(ck // L):
                heads = hidx[pl.ds(i * L, L)]
                s16 = sbuf[pl.ds(i * L, L)]
                mh = plsc.load_gather(mvec, [heads])
                ex16 = jnp.exp(s16 - mh)
                exbuf[pl.ds(i * L, L)] = ex16
                plsc.addupdate_scatter(ztab, [heads], ex16)
            pltpu.sync_copy(exbuf, ex_out.at[pl.ds(off, ck)])

    _sc_merge_tables(ztab, spm, mergebuf, accv, ztab_out, npad, jnp.add)


def _k2(s, head, mtab, npad):
    nedge = head.shape[0]
    sl = npad // NS
    kfn = pl.kernel(
        functools.partial(_k2_body, nedge, npad),
        out_type=(jax.ShapeDtypeStruct((nedge,), f32),
                  jax.ShapeDtypeStruct((NC, npad), f32)),
        mesh=_mesh,
        compiler_params=_params,
        scratch_types=[
            pltpu.VMEM((CHUNK,), i32),
            pltpu.VMEM((CHUNK,), f32),
            pltpu.VMEM((CHUNK,), f32),
            pltpu.VMEM((npad,), f32),
            pltpu.VMEM((npad,), f32),
            pltpu.VMEM((npad,), f32),
            pltpu.VMEM((NS, sl), f32),
            pltpu.VMEM((sl,), f32),
            pltpu.VMEM_SHARED((NS, npad), f32),
            pltpu.SemaphoreType.DMA,
        ],
    )
    return kfn(s, head, mtab)


def _agg_body(nedge, npad, with_attn, *refs):
    ck = ACHUNK
    if with_attn:
        (emb, head, tail, ex_in, ztab_in, agg_out,
         hidx0, hidx1, tidx0, tidx1, wbuf0, wbuf1, sidx,
         zvec, trows0, trows1, orows, spm,
         semI0, semI1, semR0, semR1, semS) = refs
    else:
        (emb, head, tail, ex_in, agg_out,
         hidx0, hidx1, tidx0, tidx1, wbuf0, wbuf1, sidx,
         trows0, trows1, orows, spm,
         semI0, semI1, semR0, semR1, semS) = refs
    hidx = (hidx0, hidx1)
    tidx = (tidx0, tidx1)
    wbuf = (wbuf0, wbuf1)
    trows = (trows0, trows1)
    semI = (semI0, semI1)
    semR = (semR0, semR1)

    nch = nedge // ck
    tpw = pl.cdiv(nch, NW)
    T = tpw + (tpw % 2)
    wid = _worker_id()
    scid = lax.axis_index("c")
    sid = lax.axis_index("s")
    sl = npad // NS
    lanes = jnp.arange(L, dtype=i32)

    if with_attn:
        pltpu.sync_copy(ztab_in.at[0], zvec)

        @pl.loop(0, npad // ck)
        def _(k):
            pltpu.sync_copy(ztab_in.at[1, pl.ds(k * ck, ck)], wbuf0)
            for cv in range(ck // L):
                d = pl.ds(k * ck + cv * L, L)
                zvec[d] = zvec[d] + wbuf0[pl.ds(cv * L, L)]

    _zero_2d(orows, ck, 128)
    for k in range(sl // ck):
        pltpu.sync_copy(orows, spm.at[pl.ds(sid * sl + k * ck, ck)])
    plsc.subcore_barrier()

    def chunk_of(g):
        c = wid + g * NW
        return jnp.minimum(c, nch - 1), c < nch

    def issue_idx(g, b):
        c, _ = chunk_of(g)
        off = c * ck
        pltpu.async_copy(head.at[pl.ds(off, ck)], hidx[b], semI[b])
        pltpu.async_copy(tail.at[pl.ds(off, ck)], tidx[b], semI[b])
        pltpu.async_copy(ex_in.at[pl.ds(off, ck)], wbuf[b], semI[b])

    def drain_idx(b):
        pltpu.make_async_copy(head.at[pl.ds(0, ck)], hidx[b], semI[b]).wait()
        pltpu.make_async_copy(tail.at[pl.ds(0, ck)], tidx[b], semI[b]).wait()
        pltpu.make_async_copy(ex_in.at[pl.ds(0, ck)], wbuf[b], semI[b]).wait()

    def issue_rows(b):
        pltpu.async_copy(emb.at[tidx[b]], trows[b], semR[b])

    def drain_rows(b):
        pltpu.make_async_copy(emb.at[tidx[b]], trows[b], semR[b]).wait()

    def drain_scatter():
        pltpu.make_async_copy(orows, spm.at[sidx], semS).wait()

    def compute(g, b):
        c, valid = chunk_of(g)
        prev_valid = jnp.logical_and(g >= 1, (wid + (g - 1) * NW) < nch)

        @pl.when(prev_valid)
        def _():
            drain_scatter()

        @pl.when(valid)
        def _():
            for j in range(ck // L):
                s = pl.ds(j * L, L)
                sidx[s] = hidx[b][s]

            for i in range(ck // L):
                io = i * L
                w16 = wbuf[b][pl.ds(io, L)]
                if with_attn:
                    heads = hidx[b][pl.ds(io, L)]
                    zh = plsc.load_gather(zvec, [heads])
                    w16 = w16 / zh
                for j in range(L):
                    e = io + j
                    a = w16[j]
                    for cv in range(128 // L):
                        cs = pl.ds(cv * L, L)
                        orows[e, cs] = trows[b][e, cs] * a

            pltpu.async_copy(orows, spm.at[sidx], semS, add=True)

    issue_idx(0, 0)
    drain_idx(0)
    issue_rows(0)
    issue_idx(1, 1)

    @pl.loop(0, T, step=2)
    def _(g2):
        for b in range(2):
            g = g2 + b
            drain_rows(b)
            drain_idx(1 - b)
            issue_rows(1 - b)
            compute(g, b)
            issue_idx(g + 2, b)

    drain_rows(0)
    drain_idx(1)
    last_valid = (wid + (T - 1) * NW) < nch

    @pl.when(last_valid)
    def _():
        drain_scatter()

    plsc.subcore_barrier()
    pltpu.sync_copy(spm.at[pl.ds(sid * sl, sl)],
                    agg_out.at[scid, pl.ds(sid * sl, sl)])


def _k3(emb, head, tail, ex, ztab, npad):
    nedge = head.shape[0]
    kfn = pl.kernel(
        functools.partial(_agg_body, nedge, npad, True),
        out_type=jax.ShapeDtypeStruct((NC, npad, 128), f32),
        mesh=_mesh,
        compiler_params=_params,
        scratch_types=[
            pltpu.VMEM((ACHUNK,), i32), pltpu.VMEM((ACHUNK,), i32),
            pltpu.VMEM((ACHUNK,), i32), pltpu.VMEM((ACHUNK,), i32),
            pltpu.VMEM((ACHUNK,), f32), pltpu.VMEM((ACHUNK,), f32),
            pltpu.VMEM((ACHUNK,), i32),
            pltpu.VMEM((npad,), f32),
            pltpu.VMEM((ACHUNK, 128), f32), pltpu.VMEM((ACHUNK, 128), f32),
            pltpu.VMEM((ACHUNK, 128), f32),
            pltpu.VMEM_SHARED((npad, 128), f32),
            pltpu.SemaphoreType.DMA, pltpu.SemaphoreType.DMA,
            pltpu.SemaphoreType.DMA, pltpu.SemaphoreType.DMA,
            pltpu.SemaphoreType.DMA,
        ],
    )
    return kfn(emb, head, tail, ex, ztab)


def _k5(emb, uidx, iidx, w, npad):
    nedge = uidx.shape[0]
    kfn = pl.kernel(
        functools.partial(_agg_body, nedge, npad, False),
        out_type=jax.ShapeDtypeStruct((NC, npad, 128), f32),
        mesh=_mesh,
        compiler_params=_params,
        scratch_types=[
            pltpu.VMEM((ACHUNK,), i32), pltpu.VMEM((ACHUNK,), i32),
            pltpu.VMEM((ACHUNK,), i32), pltpu.VMEM((ACHUNK,), i32),
            pltpu.VMEM((ACHUNK,), f32), pltpu.VMEM((ACHUNK,), f32),
            pltpu.VMEM((ACHUNK,), i32),
            pltpu.VMEM((ACHUNK, 128), f32), pltpu.VMEM((ACHUNK, 128), f32),
            pltpu.VMEM((ACHUNK, 128), f32),
            pltpu.VMEM_SHARED((npad, 128), f32),
            pltpu.SemaphoreType.DMA, pltpu.SemaphoreType.DMA,
            pltpu.SemaphoreType.DMA, pltpu.SemaphoreType.DMA,
            pltpu.SemaphoreType.DMA,
        ],
    )
    return kfn(emb, uidx, iidx, w)


def _norm_body(t_ref, o_ref):
    x = t_ref[0] + t_ref[1]
    n = jnp.sum(x * x, axis=1, keepdims=True)
    o_ref[...] = x / jnp.maximum(jnp.sqrt(n), 1e-12)


def _sum_norm(tabs, nrows):
    r = 200 if nrows % 200 == 0 else 8
    return pl.pallas_call(
        _norm_body,
        grid=(nrows // r,),
        in_specs=[pl.BlockSpec((NC, r, 128), lambda i: (0, i, 0))],
        out_specs=pl.BlockSpec((r, 128), lambda i: (i, 0)),
        out_shape=jax.ShapeDtypeStruct((nrows, 128), f32),
    )(tabs)


def kernel(user_emb, item_emb, edge_index, edge_type, inter_edge,
           inter_edge_w, relation_emb):
    n_ent = item_emb.shape[0]
    n_users = user_emb.shape[0]
    npad_e = pl.cdiv(n_ent + 1, NS * L) * NS * L

    head = edge_index[0]
    tail = edge_index[1]
    etype = edge_type
    ne = head.shape[0]
    ne_pad = pl.cdiv(ne, CHUNK) * CHUNK
    if ne_pad != ne:
        head = jnp.pad(head, (0, ne_pad - ne), constant_values=n_ent)
        tail = jnp.pad(tail, (0, ne_pad - ne), constant_values=0)
        etype = jnp.pad(etype, (0, ne_pad - ne), constant_values=0)

    emb = item_emb
    for _ in range(2):
        s, mtab = _k1(emb, head, tail, etype, relation_emb, npad_e)
        ex, ztab = _k2(s, head, mtab, npad_e)
        agg = _k3(emb, head, tail, ex, ztab, npad_e)
        emb = _sum_norm(agg, n_ent)

    uidx = inter_edge[0]
    iidx = inter_edge[1]
    w = inter_edge_w
    ni = uidx.shape[0]
    ni_pad = pl.cdiv(ni, CHUNK) * CHUNK
    npad_u = pl.cdiv(n_users + 1, NS * L) * NS * L
    if ni_pad != ni:
        uidx = jnp.pad(uidx, (0, ni_pad - ni), constant_values=n_users)
        iidx = jnp.pad(iidx, (0, ni_pad - ni), constant_values=0)
        w = jnp.pad(w, (0, ni_pad - ni), constant_values=0.0)

    uagg = _k5(emb, uidx, iidx, w, npad_u)
    user_out = _sum_norm(uagg, n_users)
    return (user_out, emb)


# speedup vs baseline: 1.0602x; 1.0081x over previous
import functools

import jax
import jax.numpy as jnp
from jax import lax
from jax.experimental import pallas as pl
from jax.experimental.pallas import tpu as pltpu
from jax.experimental.pallas import tpu_sc as plsc

f32 = jnp.float32
i32 = jnp.int32

NC = 2
NS = 16
NW = NC * NS
L = 16
CHUNK = 128
ACHUNK = 64

_mesh = plsc.VectorSubcoreMesh(core_axis_name="c", subcore_axis_name="s")
_params = pltpu.CompilerParams(needs_layout_passes=False)


def _fill_1d(ref, n, value, dtype):
    v = jnp.full((L,), value, dtype)

    @pl.loop(0, n // L)
    def _(j):
        ref[pl.ds(j * L, L)] = v


def _zero_2d(ref, rows, cols):
    z = jnp.zeros((L,), f32)

    @pl.loop(0, rows)
    def _(r):
        for cv in range(cols // L):
            ref[r, pl.ds(cv * L, L)] = z


def _scatter_max(tab, idx16, val16):
    cur = plsc.load_gather(tab, [idx16])
    new = jnp.maximum(cur, val16)
    plsc.store_scatter(tab, [idx16], new)
    chk = plsc.load_gather(tab, [idx16])
    n0 = plsc.all_reduce_population_count(chk < new)[0]

    def cond(n):
        return n > 0

    def body(n):
        chk = plsc.load_gather(tab, [idx16])
        need = chk < new
        plsc.store_scatter(tab, [idx16], new, mask=need)
        chk2 = plsc.load_gather(tab, [idx16])
        return plsc.all_reduce_population_count(chk2 < new)[0]

    lax.while_loop(cond, body, n0)


def _sc_merge_tables(part, spm, mergebuf, accv, out, npad, op):
    scid = lax.axis_index("c")
    sid = lax.axis_index("s")
    sl = npad // NS
    pltpu.sync_copy(part, spm.at[sid])
    plsc.subcore_barrier()
    for k in range(NS):
        pltpu.sync_copy(spm.at[k, pl.ds(sid * sl, sl)], mergebuf.at[k])

    @pl.loop(0, sl // L)
    def _(j):
        s = pl.ds(j * L, L)
        m = mergebuf[0, s]
        for k in range(1, NS):
            m = op(m, mergebuf[k, s])
        accv[s] = m

    pltpu.sync_copy(accv, out.at[scid, pl.ds(sid * sl, sl)])


def _worker_id():
    return lax.axis_index("s") * NC + lax.axis_index("c")


def _k1_body(nedge, npad, nrel,
             emb, head, tail, etype, rel,
             s_out, mtab_out,
             hidx0, hidx1, tidx0, tidx1, et0, et1, sbuf,
             hrows0, hrows1, trows0, trows1, relv,
             mtab, mergebuf, accv, spm,
             semI0, semI1, semR0, semR1):
    ck = CHUNK
    nch = nedge // ck
    tpw = pl.cdiv(nch, NW)
    T = tpw + (tpw % 2)
    wid = _worker_id()
    hidx = (hidx0, hidx1)
    tidx = (tidx0, tidx1)
    et = (et0, et1)
    hrows = (hrows0, hrows1)
    trows = (trows0, trows1)
    semI = (semI0, semI1)
    semR = (semR0, semR1)

    pltpu.sync_copy(rel, relv)
    _fill_1d(mtab, npad, -jnp.inf, f32)
    lanes = jnp.arange(L, dtype=i32)

    def chunk_of(g):
        c = wid + g * NW
        return jnp.minimum(c, nch - 1), c < nch

    def issue_idx(g, b):
        c, _ = chunk_of(g)
        off = c * ck
        pltpu.async_copy(head.at[pl.ds(off, ck)], hidx[b], semI[b])
        pltpu.async_copy(tail.at[pl.ds(off, ck)], tidx[b], semI[b])
        pltpu.async_copy(etype.at[pl.ds(off, ck)], et[b], semI[b])

    def drain_idx(b):
        pltpu.make_async_copy(head.at[pl.ds(0, ck)], hidx[b], semI[b]).wait()
        pltpu.make_async_copy(tail.at[pl.ds(0, ck)], tidx[b], semI[b]).wait()
        pltpu.make_async_copy(etype.at[pl.ds(0, ck)], et[b], semI[b]).wait()

    def issue_rows(b):
        pltpu.async_copy(emb.at[hidx[b]], hrows[b], semR[b])
        pltpu.async_copy(emb.at[tidx[b]], trows[b], semR[b])

    def drain_rows(b):
        pltpu.make_async_copy(emb.at[hidx[b]], hrows[b], semR[b]).wait()
        pltpu.make_async_copy(emb.at[tidx[b]], trows[b], semR[b]).wait()

    def compute(g, b):
        c, valid = chunk_of(g)

        @pl.when(valid)
        def _():
            @pl.loop(0, ck // L)
            def _(i):
                io = i * L
                heads = hidx[b][pl.ds(io, L)]
                et16 = et[b][pl.ds(io, L)]
                rrow = jnp.where(et16 == 0, nrel - 1, et16 - 1)
                s16 = jnp.zeros((L,), f32)
                for j in range(L):
                    e = io + j
                    rr = rrow[j]
                    acc = jnp.zeros((L,), f32)
                    for cv in range(128 // L):
                        cs = pl.ds(cv * L, L)
                        acc = acc + (hrows[b][e, cs] * relv[rr, cs]
                                     * trows[b][e, cs])
                    d = jnp.sum(acc)
                    s16 = jnp.where(lanes == j, d, s16)
                sv = jnp.exp(s16)
                sbuf[pl.ds(io, L)] = sv
                _scatter_max(mtab, heads, sv)

            pltpu.sync_copy(sbuf, s_out.at[pl.ds(c * ck, ck)])

    issue_idx(0, 0)
    drain_idx(0)
    issue_rows(0)
    issue_idx(1, 1)

    @pl.loop(0, T, step=2)
    def _(g2):
        for b in range(2):
            g = g2 + b
            drain_rows(b)
            drain_idx(1 - b)
            issue_rows(1 - b)
            compute(g, b)
            issue_idx(g + 2, b)

    drain_rows(0)
    drain_idx(1)

    _sc_merge_tables(mtab, spm, mergebuf, accv, mtab_out, npad, jnp.maximum)


def _k1(emb, head, tail, etype, rel, npad):
    nedge = head.shape[0]
    nrel = rel.shape[0]
    sl = npad // NS
    kfn = pl.kernel(
        functools.partial(_k1_body, nedge, npad, nrel),
        out_type=(jax.ShapeDtypeStruct((nedge,), f32),
                  jax.ShapeDtypeStruct((NC, npad), f32)),
        mesh=_mesh,
        compiler_params=_params,
        scratch_types=[
            pltpu.VMEM((CHUNK,), i32), pltpu.VMEM((CHUNK,), i32),
            pltpu.VMEM((CHUNK,), i32), pltpu.VMEM((CHUNK,), i32),
            pltpu.VMEM((CHUNK,), i32), pltpu.VMEM((CHUNK,), i32),
            pltpu.VMEM((CHUNK,), f32),
            pltpu.VMEM((CHUNK, 128), f32), pltpu.VMEM((CHUNK, 128), f32),
            pltpu.VMEM((CHUNK, 128), f32), pltpu.VMEM((CHUNK, 128), f32),
            pltpu.VMEM((nrel, 128), f32),
            pltpu.VMEM((npad,), f32),
            pltpu.VMEM((NS, sl), f32),
            pltpu.VMEM((sl,), f32),
            pltpu.VMEM_SHARED((NS, npad), f32),
            pltpu.SemaphoreType.DMA, pltpu.SemaphoreType.DMA,
            pltpu.SemaphoreType.DMA, pltpu.SemaphoreType.DMA,
        ],
    )
    return kfn(emb, head, tail, etype, rel)


def _k2_body(nedge, npad,
             s_in, head, mtab_in,
             ex_out, ztab_out,
             hidx, sbuf, exbuf, mvec, tmpv, ztab, mergebuf, accv, spm, sem):
    ck = CHUNK
    nch = nedge // ck
    tpw = pl.cdiv(nch, NW)
    wid = _worker_id()
    pltpu.sync_copy(mtab_in.at[0], mvec)
    pltpu.sync_copy(mtab_in.at[1], tmpv)

    @pl.loop(0, npad // L)
    def _(j):
        s = pl.ds(j * L, L)
        mvec[s] = jnp.maximum(mvec[s], tmpv[s])

    _fill_1d(ztab, npad, 0.0, f32)

    @pl.loop(0, tpw)
    def _(t):
        c = wid + t * NW

        @pl.when(c < nch)
        def _():
            off = c * ck
            pltpu.sync_copy(head.at[pl.ds(off, ck)], hidx)
            pltpu.sync_copy(s_in.at[pl.ds(off, ck)], sbuf)
            for i in range(ck // L):
                heads = hidx[pl.ds(i * L, L)]
                s16 = sbuf[pl.ds(i * L, L)]
                mh = plsc.load_gather(mvec, [heads])
                ex16 = jnp.exp(s16 - mh)
                exbuf[pl.ds(i * L, L)] = ex16
                plsc.addupdate_scatter(ztab, [heads], ex16)
            pltpu.sync_copy(exbuf, ex_out.at[pl.ds(off, ck)])

    _sc_merge_tables(ztab, spm, mergebuf, accv, ztab_out, npad, jnp.add)


def _k2(s, head, mtab, npad):
    nedge = head.shape[0]
    sl = npad // NS
    kfn = pl.kernel(
        functools.partial(_k2_body, nedge, npad),
        out_type=(jax.ShapeDtypeStruct((nedge,), f32),
                  jax.ShapeDtypeStruct((NC, npad), f32)),
        mesh=_mesh,
        compiler_params=_params,
        scratch_types=[
            pltpu.VMEM((CHUNK,), i32),
            pltpu.VMEM((CHUNK,), f32),
            pltpu.VMEM((CHUNK,), f32),
            pltpu.VMEM((npad,), f32),
            pltpu.VMEM((npad,), f32),
            pltpu.VMEM((npad,), f32),
            pltpu.VMEM((NS, sl), f32),
            pltpu.VMEM((sl,), f32),
            pltpu.VMEM_SHARED((NS, npad), f32),
            pltpu.SemaphoreType.DMA,
        ],
    )
    return kfn(s, head, mtab)


def _agg_body(nedge, npad, with_attn, *refs):
    ck = ACHUNK
    if with_attn:
        (emb, head, tail, ex_in, ztab_in, agg_out,
         hidx0, hidx1, tidx0, tidx1, wbuf0, wbuf1, sidx,
         zvec, trows0, trows1, orows, spm,
         semI0, semI1, semR0, semR1, semS) = refs
    else:
        (emb, head, tail, ex_in, agg_out,
         hidx0, hidx1, tidx0, tidx1, wbuf0, wbuf1, sidx,
         trows0, trows1, orows, spm,
         semI0, semI1, semR0, semR1, semS) = refs
    hidx = (hidx0, hidx1)
    tidx = (tidx0, tidx1)
    wbuf = (wbuf0, wbuf1)
    trows = (trows0, trows1)
    semI = (semI0, semI1)
    semR = (semR0, semR1)

    nch = nedge // ck
    tpw = pl.cdiv(nch, NW)
    T = tpw + (tpw % 2)
    wid = _worker_id()
    scid = lax.axis_index("c")
    sid = lax.axis_index("s")
    sl = npad // NS
    lanes = jnp.arange(L, dtype=i32)

    if with_attn:
        pltpu.sync_copy(ztab_in.at[0], zvec)

        @pl.loop(0, npad // ck)
        def _(k):
            pltpu.sync_copy(ztab_in.at[1, pl.ds(k * ck, ck)], wbuf0)
            for cv in range(ck // L):
                d = pl.ds(k * ck + cv * L, L)
                zvec[d] = zvec[d] + wbuf0[pl.ds(cv * L, L)]

    _zero_2d(orows, ck, 128)
    for k in range(sl // ck):
        pltpu.sync_copy(orows, spm.at[pl.ds(sid * sl + k * ck, ck)])
    plsc.subcore_barrier()

    def chunk_of(g):
        c = wid + g * NW
        return jnp.minimum(c, nch - 1), c < nch

    def issue_idx(g, b):
        c, _ = chunk_of(g)
        off = c * ck
        pltpu.async_copy(head.at[pl.ds(off, ck)], hidx[b], semI[b])
        pltpu.async_copy(tail.at[pl.ds(off, ck)], tidx[b], semI[b])
        pltpu.async_copy(ex_in.at[pl.ds(off, ck)], wbuf[b], semI[b])

    def drain_idx(b):
        pltpu.make_async_copy(head.at[pl.ds(0, ck)], hidx[b], semI[b]).wait()
        pltpu.make_async_copy(tail.at[pl.ds(0, ck)], tidx[b], semI[b]).wait()
        pltpu.make_async_copy(ex_in.at[pl.ds(0, ck)], wbuf[b], semI[b]).wait()

    def issue_rows(b):
        pltpu.async_copy(emb.at[tidx[b]], trows[b], semR[b])

    def drain_rows(b):
        pltpu.make_async_copy(emb.at[tidx[b]], trows[b], semR[b]).wait()

    def drain_scatter():
        pltpu.make_async_copy(orows, spm.at[sidx], semS).wait()

    def compute(g, b):
        c, valid = chunk_of(g)
        prev_valid = jnp.logical_and(g >= 1, (wid + (g - 1) * NW) < nch)

        @pl.when(prev_valid)
        def _():
            drain_scatter()

        @pl.when(valid)
        def _():
            for j in range(ck // L):
                s = pl.ds(j * L, L)
                sidx[s] = hidx[b][s]

            for i in range(ck // L):
                io = i * L
                w16 = wbuf[b][pl.ds(io, L)]
                if with_attn:
                    heads = hidx[b][pl.ds(io, L)]
                    zh = plsc.load_gather(zvec, [heads])
                    w16 = w16 / zh
                for j in range(L):
                    e = io + j
                    a = w16[j]
                    for cv in range(128 // L):
                        cs = pl.ds(cv * L, L)
                        orows[e, cs] = trows[b][e, cs] * a

            pltpu.async_copy(orows, spm.at[sidx], semS, add=True)

    issue_idx(0, 0)
    drain_idx(0)
    issue_rows(0)
    issue_idx(1, 1)

    @pl.loop(0, T, step=2)
    def _(g2):
        for b in range(2):
            g = g2 + b
            drain_rows(b)
            drain_idx(1 - b)
            issue_rows(1 - b)
            compute(g, b)
            issue_idx(g + 2, b)

    drain_rows(0)
    drain_idx(1)
    last_valid = (wid + (T - 1) * NW) < nch

    @pl.when(last_valid)
    def _():
        drain_scatter()

    plsc.subcore_barrier()
    pltpu.sync_copy(spm.at[pl.ds(sid * sl, sl)],
                    agg_out.at[scid, pl.ds(sid * sl, sl)])


def _k3(emb, head, tail, ex, ztab, npad):
    nedge = head.shape[0]
    kfn = pl.kernel(
        functools.partial(_agg_body, nedge, npad, True),
        out_type=jax.ShapeDtypeStruct((NC, npad, 128), f32),
        mesh=_mesh,
        compiler_params=_params,
        scratch_types=[
            pltpu.VMEM((ACHUNK,), i32), pltpu.VMEM((ACHUNK,), i32),
            pltpu.VMEM((ACHUNK,), i32), pltpu.VMEM((ACHUNK,), i32),
            pltpu.VMEM((ACHUNK,), f32), pltpu.VMEM((ACHUNK,), f32),
            pltpu.VMEM((ACHUNK,), i32),
            pltpu.VMEM((npad,), f32),
            pltpu.VMEM((ACHUNK, 128), f32), pltpu.VMEM((ACHUNK, 128), f32),
            pltpu.VMEM((ACHUNK, 128), f32),
            pltpu.VMEM_SHARED((npad, 128), f32),
            pltpu.SemaphoreType.DMA, pltpu.SemaphoreType.DMA,
            pltpu.SemaphoreType.DMA, pltpu.SemaphoreType.DMA,
            pltpu.SemaphoreType.DMA,
        ],
    )
    return kfn(emb, head, tail, ex, ztab)


def _k5(emb, uidx, iidx, w, npad):
    nedge = uidx.shape[0]
    kfn = pl.kernel(
        functools.partial(_agg_body, nedge, npad, False),
        out_type=jax.ShapeDtypeStruct((NC, npad, 128), f32),
        mesh=_mesh,
        compiler_params=_params,
        scratch_types=[
            pltpu.VMEM((ACHUNK,), i32), pltpu.VMEM((ACHUNK,), i32),
            pltpu.VMEM((ACHUNK,), i32), pltpu.VMEM((ACHUNK,), i32),
            pltpu.VMEM((ACHUNK,), f32), pltpu.VMEM((ACHUNK,), f32),
            pltpu.VMEM((ACHUNK,), i32),
            pltpu.VMEM((ACHUNK, 128), f32), pltpu.VMEM((ACHUNK, 128), f32),
            pltpu.VMEM((ACHUNK, 128), f32),
            pltpu.VMEM_SHARED((npad, 128), f32),
            pltpu.SemaphoreType.DMA, pltpu.SemaphoreType.DMA,
            pltpu.SemaphoreType.DMA, pltpu.SemaphoreType.DMA,
            pltpu.SemaphoreType.DMA,
        ],
    )
    return kfn(emb, uidx, iidx, w)


def _norm_body(t_ref, o_ref):
    x = t_ref[0] + t_ref[1]
    n = jnp.sum(x * x, axis=1, keepdims=True)
    o_ref[...] = x / jnp.maximum(jnp.sqrt(n), 1e-12)


def _sum_norm(tabs, nrows):
    r = 200 if nrows % 200 == 0 else 8
    return pl.pallas_call(
        _norm_body,
        grid=(nrows // r,),
        in_specs=[pl.BlockSpec((NC, r, 128), lambda i: (0, i, 0))],
        out_specs=pl.BlockSpec((r, 128), lambda i: (i, 0)),
        out_shape=jax.ShapeDtypeStruct((nrows, 128), f32),
    )(tabs)


def kernel(user_emb, item_emb, edge_index, edge_type, inter_edge,
           inter_edge_w, relation_emb):
    n_ent = item_emb.shape[0]
    n_users = user_emb.shape[0]
    npad_e = pl.cdiv(n_ent + 1, NS * L) * NS * L

    head = edge_index[0]
    tail = edge_index[1]
    etype = edge_type
    ne = head.shape[0]
    ne_pad = pl.cdiv(ne, CHUNK) * CHUNK
    if ne_pad != ne:
        head = jnp.pad(head, (0, ne_pad - ne), constant_values=n_ent)
        tail = jnp.pad(tail, (0, ne_pad - ne), constant_values=0)
        etype = jnp.pad(etype, (0, ne_pad - ne), constant_values=0)

    emb = item_emb
    for _ in range(2):
        s, mtab = _k1(emb, head, tail, etype, relation_emb, npad_e)
        ex, ztab = _k2(s, head, mtab, npad_e)
        agg = _k3(emb, head, tail, ex, ztab, npad_e)
        emb = _sum_norm(agg, n_ent)

    uidx = inter_edge[0]
    iidx = inter_edge[1]
    w = inter_edge_w
    ni = uidx.shape[0]
    ni_pad = pl.cdiv(ni, CHUNK) * CHUNK
    npad_u = pl.cdiv(n_users + 1, NS * L) * NS * L
    if ni_pad != ni:
        uidx = jnp.pad(uidx, (0, ni_pad - ni), constant_values=n_users)
        iidx = jnp.pad(iidx, (0, ni_pad - ni), constant_values=0)
        w = jnp.pad(w, (0, ni_pad - ni), constant_values=0.0)

    uagg = _k5(emb, uidx, iidx, w, npad_u)
    user_out = _sum_norm(uagg, n_users)
    return (user_out, emb)
